# baseline (speedup 1.0000x reference)
import jax
import jax.numpy as jnp
from jax import lax
from jax.experimental import pallas as pl
from jax.experimental.pallas import tpu as pltpu
from jax.experimental.pallas import tpu_sc as plsc

N = 100000
B = 128
K = 32
F = 7
NW = 32
BPW = B // NW
CROWS = 4000
CR8 = CROWS * 5 // 8
NCH = N // CROWS
BLK = 5
BLKPTS = BLK * 16
BLOCKS = CROWS // BLKPTS
IBUF_STRIDE = 64
GAMMA2 = 1.1 * 1.1


def _splat(buf2d, r, c):
    idx = jnp.full((16,), r, jnp.int32), jnp.full((16,), c, jnp.int32)
    return plsc.load_gather(buf2d, list(idx))


def _sc_body(pts_hbm, boxes_hbm, out_hbm,
             pbuf, bbuf, ibuf, gbuf, featbuf, outbuf, dsem):
    wid = lax.axis_index("c") * 16 + lax.axis_index("s")
    iota = lax.iota(jnp.int32, 16)

    pltpu.sync_copy(boxes_hbm, bbuf)
    base_box = wid * BPW
    cxs, cys, r2s, vxs, vys = [], [], [], [], []
    for j in range(BPW):
        r = base_box + j
        cxs.append(_splat(bbuf, r, 0))
        cys.append(_splat(bbuf, r, 1))
        hx = _splat(bbuf, r, 3) * jnp.float32(0.5)
        hy = _splat(bbuf, r, 4) * jnp.float32(0.5)
        r2s.append((hx * hx + hy * hy) * jnp.float32(GAMMA2))
        vxs.append(_splat(bbuf, r, 7))
        vys.append(_splat(bbuf, r, 8))

    kv = jnp.full((16,), K, jnp.int32)

    def not_done(cvs):
        lo = jnp.minimum(jnp.minimum(cvs[0], cvs[1]),
                         jnp.minimum(cvs[2], cvs[3]))
        return lo[0] < K

    def chunk_cond(st):
        return (st[0] < NCH) & not_done(st[2:])

    def chunk_body(st):
        c = st[0]
        off = pl.multiple_of(c * CROWS, 8)
        pltpu.sync_copy(pts_hbm.at[pl.ds(off, CROWS)], pbuf)

        def blk_cond(st):
            return (st[1] < BLOCKS) & not_done(st[2:])

        def blk_body(st):
            b = st[1]
            base_f = (c * CROWS + b * BLKPTS + iota).astype(jnp.float32)
            col0 = jnp.zeros((16,), jnp.int32)
            col1 = jnp.ones((16,), jnp.int32)
            xvs, yvs = [], []
            for s in range(BLK):
                ridx = b * BLKPTS + s * 16 + iota
                xvs.append(plsc.load_gather(pbuf, [ridx, col0]))
                yvs.append(plsc.load_gather(pbuf, [ridx, col1]))
            new = []
            for j in range(BPW):
                cv0 = st[2 + j]

                def do_box(cv=cv0, j=j):
                    cv = cv
                    for s in range(BLK):
                        dx = xvs[s] - cxs[j]
                        dy = yvs[s] - cys[j]
                        m = dx * dx + dy * dy <= r2s[j]
                        cs = plsc.cumsum(m.astype(jnp.int32))
                        pos = (jnp.minimum(cv, kv) + cs
                               + (j * IBUF_STRIDE - 1))
                        plsc.store_scatter(ibuf, [pos],
                                           base_f + jnp.float32(s * 16),
                                           mask=m)
                        cv = cv + plsc.all_reduce_population_count(m)
                    return cv

                new.append(lax.cond(cv0[0] < K, do_box, lambda cv=cv0: cv))
            return (c, b + 1) + tuple(new)

        st2 = lax.while_loop(blk_cond, blk_body,
                             (c, jnp.int32(0)) + st[2:])
        return (c + 1,) + st2[1:]

    zero = jnp.int32(0)
    zv = jnp.zeros((16,), jnp.int32)
    st = lax.while_loop(chunk_cond, chunk_body, (zero, zero) + (zv,) * BPW)
    cnts = [v[0] for v in st[2:]]

    waits = []
    for j in range(BPW):
        cj = jnp.minimum(cnts[j], K)
        for t in (0, 16):
            v = ibuf[pl.ds(j * IBUF_STRIDE + t, 16)].astype(jnp.int32)
            valid = (t + iota) < cj
            v = jnp.where(valid, v, 0)
            for u in range(16):
                k = j * K + t + u
                cp = pltpu.make_async_copy(pts_hbm.at[pl.ds(v[u], 1)],
                                           featbuf.at[pl.ds(k, 1)], dsem)
                cp.start()
                waits.append(cp)
    for cp in waits:
        cp.wait()

    zf = jnp.zeros((16,), jnp.float32)
    for j in range(BPW):
        cj = jnp.minimum(cnts[j], K)
        jcol = jnp.full((16,), j, jnp.int32)
        for t in (0, 16):
            sl = j * K + t + iota
            slot = t + iota
            valid = slot < cj
            for f in range(5):
                fcol = jnp.full((16,), f, jnp.int32)
                val = plsc.load_gather(featbuf, [sl, fcol])
                plsc.store_scatter(outbuf, [jcol, slot, fcol],
                                   jnp.where(valid, val, zf))
            v5 = jnp.full((16,), 5, jnp.int32)
            v6 = jnp.full((16,), 6, jnp.int32)
            plsc.store_scatter(outbuf, [jcol, slot, v5],
                               jnp.where(valid, vxs[j], zf))
            plsc.store_scatter(outbuf, [jcol, slot, v6],
                               jnp.where(valid, vys[j], zf))

    pltpu.sync_copy(outbuf, out_hbm.at[pl.ds(base_box, BPW)])


@jax.jit
def _voxel_sample(pts, boxes):
    mesh = plsc.VectorSubcoreMesh(core_axis_name="c", subcore_axis_name="s",
                                  num_cores=2, num_subcores=16)
    return pl.kernel(
        _sc_body,
        out_type=jax.ShapeDtypeStruct((B, K, F), jnp.float32),
        mesh=mesh,
        compiler_params=pltpu.CompilerParams(needs_layout_passes=False,
                                             use_tc_tiling_on_sc=False),
        scratch_types=[
            pltpu.VMEM((CROWS, 5), jnp.float32),
            pltpu.VMEM((B, 9), jnp.float32),
            pltpu.VMEM((BPW * IBUF_STRIDE,), jnp.float32),
            pltpu.VMEM((BPW * K,), jnp.int32),
            pltpu.VMEM((BPW * K, 5), jnp.float32),
            pltpu.VMEM((BPW, K, F), jnp.float32),
            pltpu.SemaphoreType.DMA,
        ],
    )(pts, boxes)


def kernel(points, boxes, num_sample):
    del num_sample
    return _voxel_sample(points, boxes)


# speedup vs baseline: 1.2355x; 1.2355x over previous
import jax
import jax.numpy as jnp
from jax import lax
from jax.experimental import pallas as pl
from jax.experimental.pallas import tpu as pltpu
from jax.experimental.pallas import tpu_sc as plsc

N = 100000
B = 128
K = 32
F = 7
NW = 32
BPW = B // NW
CROWS = 4000
CF = CROWS * 5
NCH = N // CROWS
BLK = 5
BLKPTS = BLK * 16
BLOCKS = CROWS // BLKPTS
IBUF_STRIDE = 64
GAMMA2 = 1.1 * 1.1


def _splat(buf2d, r, c):
    idx = jnp.full((16,), r, jnp.int32), jnp.full((16,), c, jnp.int32)
    return plsc.load_gather(buf2d, list(idx))


def _sc_body(pts_hbm, boxes_hbm, out_hbm,
             pbuf, bbuf, ibuf, featbuf, outbuf, dsem):
    wid = lax.axis_index("c") * 16 + lax.axis_index("s")
    iota = lax.iota(jnp.int32, 16)

    pltpu.sync_copy(boxes_hbm, bbuf)
    base_box = wid * BPW
    cxs, cys, r2s, vxs, vys = [], [], [], [], []
    for j in range(BPW):
        r = base_box + j
        cxs.append(_splat(bbuf, r, 0))
        cys.append(_splat(bbuf, r, 1))
        hx = _splat(bbuf, r, 3) * jnp.float32(0.5)
        hy = _splat(bbuf, r, 4) * jnp.float32(0.5)
        r2s.append((hx * hx + hy * hy) * jnp.float32(GAMMA2))
        vxs.append(_splat(bbuf, r, 7))
        vys.append(_splat(bbuf, r, 8))

    kv = jnp.full((16,), K, jnp.int32)

    def not_done(cvs):
        lo = jnp.minimum(jnp.minimum(cvs[0], cvs[1]),
                         jnp.minimum(cvs[2], cvs[3]))
        return lo[0] < K

    def chunk_cond(st):
        return (st[0] < NCH) & not_done(st[2:])

    def chunk_body(st):
        c = st[0]
        off = pl.multiple_of(c * CF, 8)
        pltpu.sync_copy(pts_hbm.at[pl.ds(off, CF)], pbuf)

        def blk_cond(st):
            return (st[1] < BLOCKS) & not_done(st[2:])

        def blk_body(st):
            b = st[1]
            base_f = (c * CROWS + b * BLKPTS + iota).astype(jnp.float32)
            xvs, yvs = [], []
            for s in range(BLK):
                t5 = (b * BLKPTS + s * 16 + iota) * 5
                xvs.append(plsc.load_gather(pbuf, [t5]))
                yvs.append(plsc.load_gather(pbuf, [t5 + 1]))
            new = []
            for j in range(BPW):
                cv0 = st[2 + j]

                def do_box(cv=cv0, j=j):
                    cv = cv
                    for s in range(BLK):
                        dx = xvs[s] - cxs[j]
                        dy = yvs[s] - cys[j]
                        m = dx * dx + dy * dy <= r2s[j]
                        cs = plsc.cumsum(m.astype(jnp.int32))
                        pos = (jnp.minimum(cv, kv) + cs
                               + (j * IBUF_STRIDE - 1))
                        plsc.store_scatter(ibuf, [pos],
                                           base_f + jnp.float32(s * 16),
                                           mask=m)
                        cv = cv + plsc.all_reduce_population_count(m)
                    return cv

                new.append(lax.cond(cv0[0] < K, do_box, lambda cv=cv0: cv))
            return (c, b + 1) + tuple(new)

        st2 = lax.while_loop(blk_cond, blk_body,
                             (c, jnp.int32(0)) + st[2:])
        return (c + 1,) + st2[1:]

    zero = jnp.int32(0)
    zv = jnp.zeros((16,), jnp.int32)
    st = lax.while_loop(chunk_cond, chunk_body, (zero, zero) + (zv,) * BPW)
    cnts = [v[0] for v in st[2:]]

    waits = []
    vrows = []
    for j in range(BPW):
        cj = jnp.minimum(cnts[j], K)
        for t in (0, 16):
            v = ibuf[pl.ds(j * IBUF_STRIDE + t, 16)].astype(jnp.int32)
            valid = (t + iota) < cj
            v = jnp.where(valid, v, 0)
            vrows.append(v)
            for u in range(16):
                k = j * K + t + u
                a8 = pl.multiple_of(((v[u] * 5) >> 3) * 8, 8)
                cp = pltpu.make_async_copy(pts_hbm.at[pl.ds(a8, 16)],
                                           featbuf.at[pl.ds(k * 16, 16)],
                                           dsem)
                cp.start()
                waits.append(cp)
    for cp in waits:
        cp.wait()

    zf = jnp.zeros((16,), jnp.float32)
    seven = jnp.full((16,), 7, jnp.int32)
    for j in range(BPW):
        cj = jnp.minimum(cnts[j], K)
        for t in (0, 16):
            v = vrows[j * 2 + t // 16]
            sl = j * K + t + iota
            slot = t + iota
            valid = slot < cj
            fbase = sl * 16 + ((v * 5) & seven)
            base7 = sl * 7
            for f in range(5):
                val = plsc.load_gather(featbuf, [fbase + f])
                plsc.store_scatter(outbuf, [base7 + f],
                                   jnp.where(valid, val, zf))
            plsc.store_scatter(outbuf, [base7 + 5],
                               jnp.where(valid, vxs[j], zf))
            plsc.store_scatter(outbuf, [base7 + 6],
                               jnp.where(valid, vys[j], zf))

    pltpu.sync_copy(outbuf, out_hbm.at[pl.ds(wid * (BPW * K * F), BPW * K * F)])


@jax.jit
def _voxel_sample(pts, boxes):
    mesh = plsc.VectorSubcoreMesh(core_axis_name="c", subcore_axis_name="s",
                                  num_cores=2, num_subcores=16)
    return pl.kernel(
        _sc_body,
        out_type=jax.ShapeDtypeStruct((B * K * F,), jnp.float32),
        mesh=mesh,
        compiler_params=pltpu.CompilerParams(needs_layout_passes=False,
                                             use_tc_tiling_on_sc=False),
        scratch_types=[
            pltpu.VMEM((CF,), jnp.float32),
            pltpu.VMEM((B, 9), jnp.float32),
            pltpu.VMEM((BPW * IBUF_STRIDE,), jnp.float32),
            pltpu.VMEM((BPW * K * 16,), jnp.float32),
            pltpu.VMEM((BPW * K * F,), jnp.float32),
            pltpu.SemaphoreType.DMA,
        ],
    )(pts, boxes)


def kernel(points, boxes, num_sample):
    del num_sample
    return _voxel_sample(points.reshape(-1), boxes).reshape(B, K, F)


# speedup vs baseline: 1.2480x; 1.0101x over previous
import jax
import jax.numpy as jnp
from jax import lax
from jax.experimental import pallas as pl
from jax.experimental.pallas import tpu as pltpu
from jax.experimental.pallas import tpu_sc as plsc

N = 100000
B = 128
K = 32
F = 7
NW = 32
BPW = B // NW
CROWS = 4000
CR8 = CROWS * 5 // 8
NCH = N // CROWS
BLK = 5
BLKPTS = BLK * 16
BLOCKS = CROWS // BLKPTS
IBUF_STRIDE = 64
GAMMA2 = 1.1 * 1.1


def _splat(buf2d, r, c):
    idx = jnp.full((16,), r, jnp.int32), jnp.full((16,), c, jnp.int32)
    return plsc.load_gather(buf2d, list(idx))


def _sc_body(pts8_hbm, boxes_hbm, out_hbm,
             pbuf, bbuf, ibuf, gbuf, featbuf, outbuf, dsem):
    wid = lax.axis_index("c") * 16 + lax.axis_index("s")
    iota = lax.iota(jnp.int32, 16)

    pltpu.sync_copy(boxes_hbm, bbuf)
    base_box = wid * BPW
    cxs, cys, r2s, vxs, vys = [], [], [], [], []
    for j in range(BPW):
        r = base_box + j
        cxs.append(_splat(bbuf, r, 0))
        cys.append(_splat(bbuf, r, 1))
        hx = _splat(bbuf, r, 3) * jnp.float32(0.5)
        hy = _splat(bbuf, r, 4) * jnp.float32(0.5)
        r2s.append((hx * hx + hy * hy) * jnp.float32(GAMMA2))
        vxs.append(_splat(bbuf, r, 7))
        vys.append(_splat(bbuf, r, 8))

    kv = jnp.full((16,), K, jnp.int32)

    def not_done(cvs):
        lo = jnp.minimum(jnp.minimum(cvs[0], cvs[1]),
                         jnp.minimum(cvs[2], cvs[3]))
        return lo[0] < K

    def chunk_cond(st):
        return (st[0] < NCH) & not_done(st[2:])

    def chunk_body(st):
        c = st[0]
        off = pl.multiple_of(c * CR8, 4)
        pltpu.sync_copy(pts8_hbm.at[pl.ds(off, CR8)], pbuf)

        def blk_cond(st):
            return (st[1] < BLOCKS) & not_done(st[2:])

        def blk_body(st):
            b = st[1]
            base_f = (c * CROWS + b * BLKPTS + iota).astype(jnp.float32)
            seven = jnp.full((16,), 7, jnp.int32)
            xvs, yvs = [], []
            for s in range(BLK):
                t5 = (b * BLKPTS + s * 16 + iota) * 5
                t6 = t5 + 1
                xvs.append(plsc.load_gather(pbuf, [t5 >> 3, t5 & seven]))
                yvs.append(plsc.load_gather(pbuf, [t6 >> 3, t6 & seven]))
            new = []
            for j in range(BPW):
                cv0 = st[2 + j]

                def do_box(cv=cv0, j=j):
                    cv = cv
                    for s in range(BLK):
                        dx = xvs[s] - cxs[j]
                        dy = yvs[s] - cys[j]
                        m = dx * dx + dy * dy <= r2s[j]
                        cs = plsc.cumsum(m.astype(jnp.int32))
                        pos = (jnp.minimum(cv, kv) + cs
                               + (j * IBUF_STRIDE - 1))
                        plsc.store_scatter(ibuf, [pos],
                                           base_f + jnp.float32(s * 16),
                                           mask=m)
                        cv = cv + plsc.all_reduce_population_count(m)
                    return cv

                new.append(lax.cond(cv0[0] < K, do_box, lambda cv=cv0: cv))
            return (c, b + 1) + tuple(new)

        st2 = lax.while_loop(blk_cond, blk_body,
                             (c, jnp.int32(0)) + st[2:])
        return (c + 1,) + st2[1:]

    zero = jnp.int32(0)
    zv = jnp.zeros((16,), jnp.int32)
    st = lax.while_loop(chunk_cond, chunk_body, (zero, zero) + (zv,) * BPW)
    cnts = [v[0] for v in st[2:]]

    qmax = jnp.full((16,), N * 5 // 8 - 1, jnp.int32)
    for j in range(BPW):
        cj = jnp.minimum(cnts[j], K)
        for t in (0, 16):
            v = ibuf[pl.ds(j * IBUF_STRIDE + t, 16)].astype(jnp.int32)
            valid = (t + iota) < cj
            v = jnp.where(valid, v, 0)
            q = (v * 5) >> 3
            sl2 = (j * K + t + iota) * 2
            plsc.store_scatter(gbuf, [sl2], q)
            plsc.store_scatter(gbuf, [sl2 + 1], jnp.minimum(q + 1, qmax))
    pltpu.async_copy(pts8_hbm.at[gbuf], featbuf, dsem).wait()

    zf = jnp.zeros((16,), jnp.float32)
    seven = jnp.full((16,), 7, jnp.int32)
    for j in range(BPW):
        cj = jnp.minimum(cnts[j], K)
        jcol = jnp.full((16,), j, jnp.int32)
        for t in (0, 16):
            v = ibuf[pl.ds(j * IBUF_STRIDE + t, 16)].astype(jnp.int32)
            slot = t + iota
            valid = slot < cj
            v = jnp.where(valid, v, 0)
            o = (v * 5) & seven
            sl2 = (j * K + t + iota) * 2
            for f in range(5):
                of = o + f
                row = sl2 + (of >> 3)
                col = of & seven
                val = plsc.load_gather(featbuf, [row, col])
                plsc.store_scatter(outbuf, [jcol, slot,
                                            jnp.full((16,), f, jnp.int32)],
                                   jnp.where(valid, val, zf))
            v5 = jnp.full((16,), 5, jnp.int32)
            v6 = jnp.full((16,), 6, jnp.int32)
            plsc.store_scatter(outbuf, [jcol, slot, v5],
                               jnp.where(valid, vxs[j], zf))
            plsc.store_scatter(outbuf, [jcol, slot, v6],
                               jnp.where(valid, vys[j], zf))

    pltpu.sync_copy(outbuf, out_hbm.at[pl.ds(base_box, BPW)])


@jax.jit
def _voxel_sample(pts8, boxes):
    mesh = plsc.VectorSubcoreMesh(core_axis_name="c", subcore_axis_name="s",
                                  num_cores=2, num_subcores=16)
    return pl.kernel(
        _sc_body,
        out_type=jax.ShapeDtypeStruct((B, K, F), jnp.float32),
        mesh=mesh,
        compiler_params=pltpu.CompilerParams(needs_layout_passes=False,
                                             use_tc_tiling_on_sc=False),
        scratch_types=[
            pltpu.VMEM((CR8, 8), jnp.float32),
            pltpu.VMEM((B, 9), jnp.float32),
            pltpu.VMEM((BPW * IBUF_STRIDE,), jnp.float32),
            pltpu.VMEM((BPW * K * 2,), jnp.int32),
            pltpu.VMEM((BPW * K * 2, 8), jnp.float32),
            pltpu.VMEM((BPW, K, F), jnp.float32),
            pltpu.SemaphoreType.DMA,
        ],
    )(pts8, boxes)


def kernel(points, boxes, num_sample):
    del num_sample
    return _voxel_sample(points.reshape(N * 5 // 8, 8), boxes)


# speedup vs baseline: 1.3394x; 1.0733x over previous
import jax
import jax.numpy as jnp
from jax import lax
from jax.experimental import pallas as pl
from jax.experimental.pallas import tpu as pltpu
from jax.experimental.pallas import tpu_sc as plsc

N = 100000
B = 128
K = 32
F = 7
NW = 32
BPW = B // NW
CROWS = 4000
CR8 = CROWS * 5 // 8
NCH = N // CROWS
BLK = 10
BLKPTS = BLK * 16
BLOCKS = CROWS // BLKPTS
IBUF_STRIDE = 64
GAMMA2 = 1.1 * 1.1


def _splat(buf2d, r, c):
    idx = jnp.full((16,), r, jnp.int32), jnp.full((16,), c, jnp.int32)
    return plsc.load_gather(buf2d, list(idx))


def _sc_body(pts8_hbm, boxes_hbm, out_hbm,
             pbuf, bbuf, ibuf, gbuf, featbuf, outbuf, dsem):
    wid = lax.axis_index("c") * 16 + lax.axis_index("s")
    iota = lax.iota(jnp.int32, 16)

    pltpu.sync_copy(boxes_hbm, bbuf)
    base_box = wid * BPW
    cxs, cys, r2s, vxs, vys = [], [], [], [], []
    for j in range(BPW):
        r = base_box + j
        cxs.append(_splat(bbuf, r, 0))
        cys.append(_splat(bbuf, r, 1))
        hx = _splat(bbuf, r, 3) * jnp.float32(0.5)
        hy = _splat(bbuf, r, 4) * jnp.float32(0.5)
        r2s.append((hx * hx + hy * hy) * jnp.float32(GAMMA2))
        vxs.append(_splat(bbuf, r, 7))
        vys.append(_splat(bbuf, r, 8))

    kv = jnp.full((16,), K, jnp.int32)

    def not_done(cvs):
        lo = jnp.minimum(jnp.minimum(cvs[0], cvs[1]),
                         jnp.minimum(cvs[2], cvs[3]))
        return lo[0] < K

    def chunk_cond(st):
        return (st[0] < NCH) & not_done(st[2:])

    def chunk_body(st):
        c = st[0]
        off = pl.multiple_of(c * CR8, 4)
        pltpu.sync_copy(pts8_hbm.at[pl.ds(off, CR8)], pbuf)

        def blk_cond(st):
            return (st[1] < BLOCKS) & not_done(st[2:])

        def blk_body(st):
            b = st[1]
            base_f = (c * CROWS + b * BLKPTS + iota).astype(jnp.float32)
            seven = jnp.full((16,), 7, jnp.int32)
            xvs, yvs = [], []
            for s in range(BLK):
                t5 = (b * BLKPTS + s * 16 + iota) * 5
                t6 = t5 + 1
                xvs.append(plsc.load_gather(pbuf, [t5 >> 3, t5 & seven]))
                yvs.append(plsc.load_gather(pbuf, [t6 >> 3, t6 & seven]))
            new = []
            for j in range(BPW):
                cv0 = st[2 + j]

                def do_box(cv=cv0, j=j):
                    cv = cv
                    for s in range(BLK):
                        dx = xvs[s] - cxs[j]
                        dy = yvs[s] - cys[j]
                        m = dx * dx + dy * dy <= r2s[j]
                        cs = plsc.cumsum(m.astype(jnp.int32))
                        pos = (jnp.minimum(cv, kv) + cs
                               + (j * IBUF_STRIDE - 1))
                        plsc.store_scatter(ibuf, [pos],
                                           base_f + jnp.float32(s * 16),
                                           mask=m)
                        cv = cv + plsc.all_reduce_population_count(m)
                    return cv

                new.append(lax.cond(cv0[0] < K, do_box, lambda cv=cv0: cv))
            return (c, b + 1) + tuple(new)

        st2 = lax.while_loop(blk_cond, blk_body,
                             (c, jnp.int32(0)) + st[2:])
        return (c + 1,) + st2[1:]

    zero = jnp.int32(0)
    zv = jnp.zeros((16,), jnp.int32)
    st = lax.while_loop(chunk_cond, chunk_body, (zero, zero) + (zv,) * BPW)
    cnts = [v[0] for v in st[2:]]

    qmax = jnp.full((16,), N * 5 // 8 - 1, jnp.int32)
    for j in range(BPW):
        cj = jnp.minimum(cnts[j], K)
        for t in (0, 16):
            v = ibuf[pl.ds(j * IBUF_STRIDE + t, 16)].astype(jnp.int32)
            valid = (t + iota) < cj
            v = jnp.where(valid, v, 0)
            q = (v * 5) >> 3
            sl2 = (j * K + t + iota) * 2
            plsc.store_scatter(gbuf, [sl2], q)
            plsc.store_scatter(gbuf, [sl2 + 1], jnp.minimum(q + 1, qmax))
    pltpu.async_copy(pts8_hbm.at[gbuf], featbuf, dsem).wait()

    zf = jnp.zeros((16,), jnp.float32)
    seven = jnp.full((16,), 7, jnp.int32)
    for j in range(BPW):
        cj = jnp.minimum(cnts[j], K)
        jcol = jnp.full((16,), j, jnp.int32)
        for t in (0, 16):
            v = ibuf[pl.ds(j * IBUF_STRIDE + t, 16)].astype(jnp.int32)
            slot = t + iota
            valid = slot < cj
            v = jnp.where(valid, v, 0)
            o = (v * 5) & seven
            sl2 = (j * K + t + iota) * 2
            for f in range(5):
                of = o + f
                row = sl2 + (of >> 3)
                col = of & seven
                val = plsc.load_gather(featbuf, [row, col])
                plsc.store_scatter(outbuf, [jcol, slot,
                                            jnp.full((16,), f, jnp.int32)],
                                   jnp.where(valid, val, zf))
            v5 = jnp.full((16,), 5, jnp.int32)
            v6 = jnp.full((16,), 6, jnp.int32)
            plsc.store_scatter(outbuf, [jcol, slot, v5],
                               jnp.where(valid, vxs[j], zf))
            plsc.store_scatter(outbuf, [jcol, slot, v6],
                               jnp.where(valid, vys[j], zf))

    pltpu.sync_copy(outbuf, out_hbm.at[pl.ds(base_box, BPW)])


@jax.jit
def _voxel_sample(pts8, boxes):
    mesh = plsc.VectorSubcoreMesh(core_axis_name="c", subcore_axis_name="s",
                                  num_cores=2, num_subcores=16)
    return pl.kernel(
        _sc_body,
        out_type=jax.ShapeDtypeStruct((B, K, F), jnp.float32),
        mesh=mesh,
        compiler_params=pltpu.CompilerParams(needs_layout_passes=False,
                                             use_tc_tiling_on_sc=False),
        scratch_types=[
            pltpu.VMEM((CR8, 8), jnp.float32),
            pltpu.VMEM((B, 9), jnp.float32),
            pltpu.VMEM((BPW * IBUF_STRIDE,), jnp.float32),
            pltpu.VMEM((BPW * K * 2,), jnp.int32),
            pltpu.VMEM((BPW * K * 2, 8), jnp.float32),
            pltpu.VMEM((BPW, K, F), jnp.float32),
            pltpu.SemaphoreType.DMA,
        ],
    )(pts8, boxes)


def kernel(points, boxes, num_sample):
    del num_sample
    return _voxel_sample(points.reshape(N * 5 // 8, 8), boxes)


# speedup vs baseline: 1.5646x; 1.1682x over previous
import jax
import jax.numpy as jnp
from jax import lax
from jax.experimental import pallas as pl
from jax.experimental.pallas import tpu as pltpu
from jax.experimental.pallas import tpu_sc as plsc

N = 100000
B = 128
K = 32
F = 7
NW = 32
BPW = B // NW
CROWS = 4000
CR8 = CROWS * 5 // 16
NCH = N // CROWS
BLK = 10
BLKPTS = BLK * 16
BLOCKS = CROWS // BLKPTS
IBUF_STRIDE = 64
GAMMA2 = 1.1 * 1.1


def _splat(buf2d, r, c):
    idx = jnp.full((16,), r, jnp.int32), jnp.full((16,), c, jnp.int32)
    return plsc.load_gather(buf2d, list(idx))


def _sc_body(pts8_hbm, boxes_hbm, out_hbm,
             pbuf, bbuf, ibuf, gbuf, featbuf, outbuf, dsem):
    wid = lax.axis_index("c") * 16 + lax.axis_index("s")
    iota = lax.iota(jnp.int32, 16)

    pltpu.sync_copy(boxes_hbm, bbuf)
    base_box = wid * BPW
    cxs, cys, r2s, vxs, vys = [], [], [], [], []
    for j in range(BPW):
        r = base_box + j
        cxs.append(_splat(bbuf, r, 0))
        cys.append(_splat(bbuf, r, 1))
        hx = _splat(bbuf, r, 3) * jnp.float32(0.5)
        hy = _splat(bbuf, r, 4) * jnp.float32(0.5)
        r2s.append((hx * hx + hy * hy) * jnp.float32(GAMMA2))
        vxs.append(_splat(bbuf, r, 7))
        vys.append(_splat(bbuf, r, 8))

    kv = jnp.full((16,), K, jnp.int32)

    def not_done(cvs):
        lo = jnp.minimum(jnp.minimum(cvs[0], cvs[1]),
                         jnp.minimum(cvs[2], cvs[3]))
        return lo[0] < K

    def chunk_cond(st):
        return (st[0] < NCH) & not_done(st[2:])

    def chunk_body(st):
        c = st[0]
        off = pl.multiple_of(c * CR8, 4)
        pltpu.sync_copy(pts8_hbm.at[pl.ds(off, CR8)], pbuf)

        def blk_cond(st):
            return (st[1] < BLOCKS) & not_done(st[2:])

        def blk_body(st):
            b = st[1]
            base_f = (c * CROWS + b * BLKPTS + iota).astype(jnp.float32)
            fifteen = jnp.full((16,), 15, jnp.int32)
            xvs, yvs = [], []
            for s in range(BLK):
                t5 = (b * BLKPTS + s * 16 + iota) * 5
                t6 = t5 + 1
                xvs.append(plsc.load_gather(pbuf, [t5 >> 4, t5 & fifteen]))
                yvs.append(plsc.load_gather(pbuf, [t6 >> 4, t6 & fifteen]))
            new = []
            for j in range(BPW):
                cv0 = st[2 + j]

                def do_box(cv=cv0, j=j):
                    cv = cv
                    for s in range(BLK):
                        dx = xvs[s] - cxs[j]
                        dy = yvs[s] - cys[j]
                        m = dx * dx + dy * dy <= r2s[j]
                        cs = plsc.cumsum(m.astype(jnp.int32))
                        pos = (jnp.minimum(cv, kv) + cs
                               + (j * IBUF_STRIDE - 1))
                        plsc.store_scatter(ibuf, [pos],
                                           base_f + jnp.float32(s * 16),
                                           mask=m)
                        cv = cv + plsc.all_reduce_population_count(m)
                    return cv

                new.append(lax.cond(cv0[0] < K, do_box, lambda cv=cv0: cv))
            return (c, b + 1) + tuple(new)

        st2 = lax.while_loop(blk_cond, blk_body,
                             (c, jnp.int32(0)) + st[2:])
        return (c + 1,) + st2[1:]

    zero = jnp.int32(0)
    zv = jnp.zeros((16,), jnp.int32)
    st = lax.while_loop(chunk_cond, chunk_body, (zero, zero) + (zv,) * BPW)
    cnts = [v[0] for v in st[2:]]

    qmax = jnp.full((16,), N * 5 // 16 - 1, jnp.int32)
    for j in range(BPW):
        cj = jnp.minimum(cnts[j], K)
        for t in (0, 16):
            v = ibuf[pl.ds(j * IBUF_STRIDE + t, 16)].astype(jnp.int32)
            valid = (t + iota) < cj
            v = jnp.where(valid, v, 0)
            q = (v * 5) >> 4
            sl2 = (j * K + t + iota) * 2
            plsc.store_scatter(gbuf, [sl2], q)
            plsc.store_scatter(gbuf, [sl2 + 1], jnp.minimum(q + 1, qmax))
    pltpu.async_copy(pts8_hbm.at[gbuf], featbuf, dsem).wait()

    zf = jnp.zeros((16,), jnp.float32)
    fifteen = jnp.full((16,), 15, jnp.int32)
    for j in range(BPW):
        cj = jnp.minimum(cnts[j], K)
        jcol = jnp.full((16,), j, jnp.int32)
        for t in (0, 16):
            v = ibuf[pl.ds(j * IBUF_STRIDE + t, 16)].astype(jnp.int32)
            slot = t + iota
            valid = slot < cj
            v = jnp.where(valid, v, 0)
            o = (v * 5) & fifteen
            sl2 = (j * K + t + iota) * 2
            for f in range(5):
                of = o + f
                row = sl2 + (of >> 4)
                col = of & fifteen
                val = plsc.load_gather(featbuf, [row, col])
                plsc.store_scatter(outbuf, [jcol, slot,
                                            jnp.full((16,), f, jnp.int32)],
                                   jnp.where(valid, val, zf))
            v5 = jnp.full((16,), 5, jnp.int32)
            v6 = jnp.full((16,), 6, jnp.int32)
            plsc.store_scatter(outbuf, [jcol, slot, v5],
                               jnp.where(valid, vxs[j], zf))
            plsc.store_scatter(outbuf, [jcol, slot, v6],
                               jnp.where(valid, vys[j], zf))

    pltpu.sync_copy(outbuf, out_hbm.at[pl.ds(base_box, BPW)])


@jax.jit
def _voxel_sample(pts8, boxes):
    mesh = plsc.VectorSubcoreMesh(core_axis_name="c", subcore_axis_name="s",
                                  num_cores=2, num_subcores=16)
    return pl.kernel(
        _sc_body,
        out_type=jax.ShapeDtypeStruct((B, K, F), jnp.float32),
        mesh=mesh,
        compiler_params=pltpu.CompilerParams(needs_layout_passes=False,
                                             use_tc_tiling_on_sc=False),
        scratch_types=[
            pltpu.VMEM((CR8, 16), jnp.float32),
            pltpu.VMEM((B, 9), jnp.float32),
            pltpu.VMEM((BPW * IBUF_STRIDE,), jnp.float32),
            pltpu.VMEM((BPW * K * 2,), jnp.int32),
            pltpu.VMEM((BPW * K * 2, 16), jnp.float32),
            pltpu.VMEM((BPW, K, F), jnp.float32),
            pltpu.SemaphoreType.DMA,
        ],
    )(pts8, boxes)


def kernel(points, boxes, num_sample):
    del num_sample
    return _voxel_sample(points.reshape(N * 5 // 16, 16), boxes)
